# baseline (device time: 124388 ns/iter reference)
import jax
import jax.numpy as jnp
from jax import lax
from jax.experimental import pallas as pl
from jax.experimental.pallas import tpu as pltpu

N_DEV = 8


def _mlp_layer(x_shard, win, wout, *, collective_id):
    m, d = x_shard.shape

    def body(x_ref, win_ref, wout_ref, out_ref,
             xg, part, ra_buf, rb_from_r, rb_from_l, rc_l, rc_r, rc_z,
             pb_buf, send_sems, recv_sems):
        my = lax.axis_index("i")
        base = (my // 4) * 4
        q = my - base
        r_in = base + lax.rem(q + 1, 4)
        l_in = base + lax.rem(q + 3, 4)
        zp = lax.rem(my + 4, N_DEV)

        def copy(k, src, dst, dev):
            return pltpu.make_async_remote_copy(
                src_ref=src, dst_ref=dst, send_sem=send_sems.at[k],
                recv_sem=recv_sems.at[k], device_id=(dev,),
                device_id_type=pl.DeviceIdType.MESH,
            )

        def xslot(k, n=1):
            return xg.at[k * m:(k + n) * m, :]

        def matmul(xb):
            h = jnp.maximum(
                jnp.dot(xb, win_ref[...],
                        preferred_element_type=jnp.float32),
                0.0)
            return jnp.dot(h, wout_ref[...],
                           preferred_element_type=jnp.float32)

        barrier = pltpu.get_barrier_semaphore()
        for nbr in (l_in, r_in, zp):
            pl.semaphore_signal(barrier, inc=1, device_id=(nbr,),
                                device_id_type=pl.DeviceIdType.MESH)
        pl.semaphore_wait(barrier, 3)

        xg[0:m, :] = x_ref[...]
        t0 = [copy(0, xslot(0), xslot(2), l_in),
              copy(1, xslot(0), xslot(1), r_in),
              copy(2, xslot(0), xslot(3), zp)]
        for rd in t0:
            rd.start()

        part[7 * m:8 * m, :] = matmul(x_ref[...])

        t0[1].wait_recv()
        t0[2].wait_recv()
        t0[0].wait_recv()

        t1 = [copy(3, xslot(1), xslot(4), r_in),
              copy(4, xslot(3), xslot(5), r_in),
              copy(5, xslot(3), xslot(6), l_in)]
        for rd in t1:
            rd.start()

        part[4 * m:7 * m, :] = matmul(xg[m:4 * m, :])
        rc_to_r = copy(11, part.at[5 * m:6 * m, :], rc_l, r_in)
        rc_to_r.start()

        t1[0].wait_recv()
        t2 = copy(6, xslot(4), xslot(7), zp)
        t2.start()

        t1[1].wait_recv()
        part[1 * m:3 * m, :] = matmul(xg[4 * m:6 * m, :])
        pb_to_l = copy(13, part.at[1 * m:2 * m, :], pb_buf, l_in)
        pb_to_l.start()

        t1[2].wait_recv()
        part[3 * m:4 * m, :] = matmul(xg[6 * m:7 * m, :])
        rb_to_r = copy(9, part.at[3 * m:4 * m, :], rb_from_l, r_in)
        rb_to_r.start()

        pb_to_l.wait_recv()
        part[4 * m:5 * m, :] = part[4 * m:5 * m, :] + pb_buf[...]
        rc_to_l = copy(10, part.at[4 * m:5 * m, :], rc_r, l_in)
        rc_to_l.start()

        t2.wait_recv()
        part[0:m, :] = matmul(xg[7 * m:8 * m, :])
        ra_to_l = copy(7, part.at[0:m, :], ra_buf, l_in)
        ra_to_l.start()

        ra_to_l.wait_recv()
        part[2 * m:3 * m, :] = part[2 * m:3 * m, :] + ra_buf[...]
        rb_to_l = copy(8, part.at[2 * m:3 * m, :], rb_from_r, l_in)
        rb_to_l.start()

        rb_to_l.wait_recv()
        part[6 * m:7 * m, :] = part[6 * m:7 * m, :] + rb_from_r[...]
        rb_to_r.wait_recv()
        part[6 * m:7 * m, :] = part[6 * m:7 * m, :] + rb_from_l[...]
        rc_to_z = copy(12, part.at[6 * m:7 * m, :], rc_z, zp)
        rc_to_z.start()

        rc_to_l.wait_recv()
        rc_to_r.wait_recv()
        rc_to_z.wait_recv()
        out_ref[...] = (
            (part[7 * m:8 * m, :] + rc_z[...]) + (rc_l[...] + rc_r[...])
        )

        for rd in t0 + t1 + [t2, pb_to_l, ra_to_l, rb_to_l, rb_to_r,
                             rc_to_l, rc_to_r, rc_to_z]:
            rd.wait_send()

    return pl.pallas_call(
        body,
        out_shape=jax.ShapeDtypeStruct((m, d), jnp.float32),
        in_specs=[pl.BlockSpec(memory_space=pltpu.VMEM)] * 3,
        out_specs=pl.BlockSpec(memory_space=pltpu.VMEM),
        scratch_shapes=[
            pltpu.VMEM((N_DEV * m, d), jnp.float32),
            pltpu.VMEM((N_DEV * m, d), jnp.float32),
            pltpu.VMEM((m, d), jnp.float32),
            pltpu.VMEM((m, d), jnp.float32),
            pltpu.VMEM((m, d), jnp.float32),
            pltpu.VMEM((m, d), jnp.float32),
            pltpu.VMEM((m, d), jnp.float32),
            pltpu.VMEM((m, d), jnp.float32),
            pltpu.VMEM((m, d), jnp.float32),
            pltpu.SemaphoreType.DMA((14,)),
            pltpu.SemaphoreType.DMA((14,)),
        ],
        compiler_params=pltpu.CompilerParams(collective_id=collective_id),
    )(x_shard, win, wout)


def kernel(x, Win0, Wout0, Win1, Wout1, Win2, Wout2):
    x = _mlp_layer(x, Win0, Wout0, collective_id=0)
    x = _mlp_layer(x, Win1, Wout1, collective_id=1)
    x = _mlp_layer(x, Win2, Wout2, collective_id=2)
    return x


# device time: 113965 ns/iter; 1.0915x vs baseline; 1.0915x over previous
import jax
import jax.numpy as jnp
from jax import lax
from jax.experimental import pallas as pl
from jax.experimental.pallas import tpu as pltpu

N_DEV = 8


def _mlp_layer(x_shard, win, wout, *, collective_id):
    m, d = x_shard.shape

    def body(x_ref, win_ref, wout_ref, out_ref,
             xg, psend, prec, own, ag_s, ag_r, rs_s, rs_r):
        my = lax.axis_index("i")

        def matmul(xb):
            h = jnp.maximum(
                jnp.dot(xb, win_ref[...], preferred_element_type=jnp.float32),
                0.0)
            return jnp.dot(h, wout_ref[...],
                           preferred_element_type=jnp.float32)

        barrier = pltpu.get_barrier_semaphore()
        for k in range(1, N_DEV):
            pl.semaphore_signal(
                barrier, inc=1,
                device_id=(lax.rem(my + k, N_DEV),),
                device_id_type=pl.DeviceIdType.MESH)
        pl.semaphore_wait(barrier, N_DEV - 1)

        my_slot = pl.ds(my * m, m)
        ag = []
        for k in range(1, N_DEV):
            rd = pltpu.make_async_remote_copy(
                src_ref=x_ref,
                dst_ref=xg.at[my_slot, :],
                send_sem=ag_s.at[k - 1], recv_sem=ag_r.at[k - 1],
                device_id=(lax.rem(my + k, N_DEV),),
                device_id_type=pl.DeviceIdType.MESH)
            rd.start()
            ag.append(rd)

        own[...] = matmul(x_ref[...])

        rs = []
        for k in range(1, N_DEV):
            ag[k - 1].wait_recv()
            blk = pl.ds(lax.rem(my - k + N_DEV, N_DEV) * m, m)
            psend[(k - 1) * m:k * m, :] = matmul(xg[blk, :])
            rd = pltpu.make_async_remote_copy(
                src_ref=psend.at[(k - 1) * m:k * m, :],
                dst_ref=prec.at[(k - 1) * m:k * m, :],
                send_sem=rs_s.at[k - 1], recv_sem=rs_r.at[k - 1],
                device_id=(lax.rem(my - k + N_DEV, N_DEV),),
                device_id_type=pl.DeviceIdType.MESH)
            rd.start()
            rs.append(rd)

        for rd in rs:
            rd.wait_recv()
        s01 = prec[0:m, :] + prec[m:2 * m, :]
        s23 = prec[2 * m:3 * m, :] + prec[3 * m:4 * m, :]
        s45 = prec[4 * m:5 * m, :] + prec[5 * m:6 * m, :]
        s67 = prec[6 * m:7 * m, :] + own[...]
        out_ref[...] = (s01 + s23) + (s45 + s67)

        for rd in ag + rs:
            rd.wait_send()

    return pl.pallas_call(
        body,
        out_shape=jax.ShapeDtypeStruct((m, d), jnp.float32),
        in_specs=[pl.BlockSpec(memory_space=pltpu.VMEM)] * 3,
        out_specs=pl.BlockSpec(memory_space=pltpu.VMEM),
        scratch_shapes=[
            pltpu.VMEM((N_DEV * m, d), jnp.float32),
            pltpu.VMEM(((N_DEV - 1) * m, d), jnp.float32),
            pltpu.VMEM(((N_DEV - 1) * m, d), jnp.float32),
            pltpu.VMEM((m, d), jnp.float32),
            pltpu.SemaphoreType.DMA((N_DEV - 1,)),
            pltpu.SemaphoreType.DMA((N_DEV - 1,)),
            pltpu.SemaphoreType.DMA((N_DEV - 1,)),
            pltpu.SemaphoreType.DMA((N_DEV - 1,)),
        ],
        compiler_params=pltpu.CompilerParams(collective_id=collective_id),
    )(x_shard, win, wout)


def kernel(x, Win0, Wout0, Win1, Wout1, Win2, Wout2):
    x = _mlp_layer(x, Win0, Wout0, collective_id=0)
    x = _mlp_layer(x, Win1, Wout1, collective_id=1)
    x = _mlp_layer(x, Win2, Wout2, collective_id=2)
    return x


# device time: 98858 ns/iter; 1.2582x vs baseline; 1.1528x over previous
import jax
import jax.numpy as jnp
from jax import lax
from jax.experimental import pallas as pl
from jax.experimental.pallas import tpu as pltpu

N_DEV = 8
N_LAYER = 3


def kernel(x, Win0, Wout0, Win1, Wout1, Win2, Wout2):
    m, d = x.shape
    dh = Win0.shape[1]

    def body(x_ref, win0, wout0, win1, wout1, win2, wout2, out_ref,
             winbuf, woutbuf, xg, psend, prec, own, xbuf,
             ag_s, ag_r, rs_s, rs_r, wsem):
        my = lax.axis_index("i")
        wins = [win0, win1, win2]
        wouts = [wout0, wout1, wout2]

        wcp = []
        for l in range(2):
            cp_in = pltpu.make_async_copy(wins[l], winbuf.at[l], wsem.at[2 * l])
            cp_out = pltpu.make_async_copy(wouts[l], woutbuf.at[l],
                                           wsem.at[2 * l + 1])
            cp_in.start()
            cp_out.start()
            wcp.append((cp_in, cp_out))

        barrier = pltpu.get_barrier_semaphore()
        for k in range(1, N_DEV):
            pl.semaphore_signal(
                barrier, inc=1,
                device_id=(lax.rem(my + k, N_DEV),),
                device_id_type=pl.DeviceIdType.MESH)
        pl.semaphore_wait(barrier, N_DEV - 1)

        my_slot = pl.ds(my * m, m)

        for l in range(N_LAYER):
            par = l % 2
            xin = x_ref if l == 0 else xbuf.at[(l - 1) % 2]

            def matmul(xb, _l=l):
                h = jnp.maximum(
                    jnp.dot(xb, winbuf[_l % 2],
                            preferred_element_type=jnp.float32),
                    0.0)
                return jnp.dot(h, woutbuf[_l % 2],
                               preferred_element_type=jnp.float32)

            ag = []
            for k in range(1, N_DEV):
                rd = pltpu.make_async_remote_copy(
                    src_ref=xin,
                    dst_ref=xg.at[pl.ds(par * N_DEV * m + my * m, m), :],
                    send_sem=ag_s.at[l, k - 1], recv_sem=ag_r.at[l, k - 1],
                    device_id=(lax.rem(my + k, N_DEV),),
                    device_id_type=pl.DeviceIdType.MESH)
                rd.start()
                ag.append(rd)

            wcp[l][0].wait()
            wcp[l][1].wait()
            own[...] = matmul(xin[...])

            if l == 0:
                pass

            rs = []
            for k in range(1, N_DEV):
                ag[k - 1].wait_recv()
                blk = pl.ds(par * N_DEV * m
                            + lax.rem(my - k + N_DEV, N_DEV) * m, m)
                po = par * (N_DEV - 1) * m + (k - 1) * m
                psend[po:po + m, :] = matmul(xg[blk, :])
                rd = pltpu.make_async_remote_copy(
                    src_ref=psend.at[po:po + m, :],
                    dst_ref=prec.at[po:po + m, :],
                    send_sem=rs_s.at[l, k - 1], recv_sem=rs_r.at[l, k - 1],
                    device_id=(lax.rem(my - k + N_DEV, N_DEV),),
                    device_id_type=pl.DeviceIdType.MESH)
                rd.start()
                rs.append(rd)

            if l == 0:
                cp_in = pltpu.make_async_copy(wins[2], winbuf.at[0],
                                              wsem.at[4])
                cp_out = pltpu.make_async_copy(wouts[2], woutbuf.at[0],
                                               wsem.at[5])
                cp_in.start()
                cp_out.start()
                wcp.append((cp_in, cp_out))

            for rd in rs:
                rd.wait_recv()
            pb = par * (N_DEV - 1) * m
            s01 = prec[pb:pb + m, :] + prec[pb + m:pb + 2 * m, :]
            s23 = prec[pb + 2 * m:pb + 3 * m, :] + prec[pb + 3 * m:pb + 4 * m, :]
            s45 = prec[pb + 4 * m:pb + 5 * m, :] + prec[pb + 5 * m:pb + 6 * m, :]
            s67 = prec[pb + 6 * m:pb + 7 * m, :] + own[...]
            result = (s01 + s23) + (s45 + s67)
            if l == N_LAYER - 1:
                out_ref[...] = result
            else:
                xbuf[l % 2, :, :] = result

            for rd in ag + rs:
                rd.wait_send()

    vmem = pl.BlockSpec(memory_space=pltpu.VMEM)
    anymem = pl.BlockSpec(memory_space=pltpu.MemorySpace.HBM)
    return pl.pallas_call(
        body,
        out_shape=jax.ShapeDtypeStruct((m, d), jnp.float32),
        in_specs=[vmem] + [anymem] * 6,
        out_specs=vmem,
        scratch_shapes=[
            pltpu.VMEM((2, d, dh), jnp.float32),
            pltpu.VMEM((2, dh, d), jnp.float32),
            pltpu.VMEM((2 * N_DEV * m, d), jnp.float32),
            pltpu.VMEM((2 * (N_DEV - 1) * m, d), jnp.float32),
            pltpu.VMEM((2 * (N_DEV - 1) * m, d), jnp.float32),
            pltpu.VMEM((m, d), jnp.float32),
            pltpu.VMEM((2, m, d), jnp.float32),
            pltpu.SemaphoreType.DMA((N_LAYER, N_DEV - 1)),
            pltpu.SemaphoreType.DMA((N_LAYER, N_DEV - 1)),
            pltpu.SemaphoreType.DMA((N_LAYER, N_DEV - 1)),
            pltpu.SemaphoreType.DMA((N_LAYER, N_DEV - 1)),
            pltpu.SemaphoreType.DMA((6,)),
        ],
        compiler_params=pltpu.CompilerParams(
            collective_id=0, vmem_limit_bytes=60 * 1024 * 1024),
    )(x, Win0, Wout0, Win1, Wout1, Win2, Wout2)
